# baseline (speedup 1.0000x reference)
import jax
import jax.numpy as jnp
from jax.experimental import pallas as pl
from jax.experimental.pallas import tpu as pltpu

_BM = 400
_NSLOT = 3


def _fused_kernel(x_ref, w_ref, nb_ref, out_ref, buf_ref, x1_ref, sems):
    i = pl.program_id(0)
    num = pl.num_programs(0)

    def start_tile(tile, slot):
        pltpu.make_async_copy(
            nb_ref.at[pl.ds(tile * _BM, _BM), :],
            buf_ref.at[slot],
            sems.at[slot],
        ).start()

    @pl.when(i == 0)
    def _():
        for t in range(_NSLOT - 1):
            start_tile(t, t)
        x1_ref[...] = jax.lax.dot_general(
            x_ref[...], w_ref[...],
            (((1,), (1,)), ((), ())),
            preferred_element_type=jnp.float32,
        ).astype(jnp.bfloat16)

    @pl.when(i + _NSLOT - 1 < num)
    def _():
        start_tile(i + _NSLOT - 1, (i + _NSLOT - 1) % _NSLOT)

    slot = i % _NSLOT
    pltpu.make_async_copy(
        nb_ref.at[pl.ds(i * _BM, _BM), :],
        buf_ref.at[slot],
        sems.at[slot],
    ).wait()

    out_ref[...] = jax.lax.dot(
        buf_ref[slot], x1_ref[...],
        preferred_element_type=jnp.float32,
    )


def kernel(x, neighborhood, W):
    n, d_in = x.shape
    d_out = W.shape[0]
    assert n % _BM == 0
    grid = (n // _BM,)
    return pl.pallas_call(
        _fused_kernel,
        grid=grid,
        in_specs=[
            pl.BlockSpec((n, d_in), lambda i: (0, 0)),
            pl.BlockSpec((d_out, d_in), lambda i: (0, 0)),
            pl.BlockSpec(memory_space=pltpu.MemorySpace.HBM),
        ],
        out_specs=pl.BlockSpec((_BM, d_out), lambda i: (i, 0)),
        out_shape=jax.ShapeDtypeStruct((n, d_out), jnp.float32),
        scratch_shapes=[
            pltpu.VMEM((_NSLOT, _BM, n), jnp.float32),
            pltpu.VMEM((n, d_out), jnp.bfloat16),
            pltpu.SemaphoreType.DMA((_NSLOT,)),
        ],
        compiler_params=pltpu.CompilerParams(
            dimension_semantics=("arbitrary",),
        ),
    )(x, W, neighborhood)


# speedup vs baseline: 1.0372x; 1.0372x over previous
import jax
import jax.numpy as jnp
from jax.experimental import pallas as pl
from jax.experimental.pallas import tpu as pltpu


def _fused_kernel(x_ref, w_ref, nb_ref, out_ref, x1_ref):
    @pl.when(pl.program_id(0) == 0)
    def _():
        x1_ref[...] = jax.lax.dot_general(
            x_ref[...], w_ref[...],
            (((1,), (1,)), ((), ())),
            preferred_element_type=jnp.float32,
        ).astype(jnp.bfloat16)

    out_ref[...] = jax.lax.dot(
        nb_ref[...], x1_ref[...],
        preferred_element_type=jnp.float32,
    )


def kernel(x, neighborhood, W):
    n, d_in = x.shape
    d_out = W.shape[0]
    bm = 400
    assert n % bm == 0
    grid = (n // bm,)
    return pl.pallas_call(
        _fused_kernel,
        grid=grid,
        in_specs=[
            pl.BlockSpec((n, d_in), lambda i: (0, 0)),
            pl.BlockSpec((d_out, d_in), lambda i: (0, 0)),
            pl.BlockSpec((bm, n), lambda i: (i, 0)),
        ],
        out_specs=pl.BlockSpec((bm, d_out), lambda i: (i, 0)),
        out_shape=jax.ShapeDtypeStruct((n, d_out), jnp.float32),
        scratch_shapes=[pltpu.VMEM((n, d_out), jnp.bfloat16)],
        compiler_params=pltpu.CompilerParams(
            dimension_semantics=("arbitrary",),
        ),
    )(x, W, neighborhood)
